# baseline (speedup 1.0000x reference)
import functools

import jax
import jax.numpy as jnp
from jax import lax
from jax.experimental import pallas as pl
from jax.experimental.pallas import tpu as pltpu
from jax.experimental.pallas import tpu_sc as plsc

NJ = 24
COL = 14
TMP = 3
P = COL * COL
B = 512
BJ = B * NJ
NTILES = 32
CHUNK = BJ // NTILES
NV = CHUNK // 16
SCALE = 1.0 / float(COL)


def _h_body(h_ref, t0_ref, t1_ref, v0_ref, idx_ref, s1_ref, acc_ref, *,
            nsteps):
    step = pl.program_id(0)

    h = h_ref[...]
    py = lax.broadcasted_iota(jnp.int32, h.shape, 0)
    px = lax.broadcasted_iota(jnp.int32, h.shape, 1)
    p = py * COL + px
    m = jnp.max(jnp.max(h, axis=0), axis=0)
    hit = jnp.where(h == m[None, None], p, P)
    idx = jnp.min(jnp.min(hit, axis=0), axis=0)
    idx_ref[...] = idx

    t0 = t0_ref[...]
    t1 = t1_ref[...]
    v0 = v0_ref[...]

    mu_x = (t0 * COL + 0.5).astype(jnp.int32)
    mu_y = (t1 * COL + 0.5).astype(jnp.int32)
    v0i = v0.astype(jnp.int32)
    oob = ((mu_x - TMP >= COL) | (mu_y - TMP >= COL)
           | (mu_x + TMP + 1 < 0) | (mu_y + TMP + 1 < 0))
    kill = (v0i == 1) & oob
    vis = (v0i == 1) & (~oob)
    vn0 = jnp.where(kill, 0.0, v0)
    mask1 = (vn0.astype(jnp.int32) != 0)
    m1f = mask1.astype(jnp.float32)
    visf = vis.astype(jnp.float32)

    dx = px - mu_x[None, None]
    dy = py - mu_y[None, None]
    dxf = dx.astype(jnp.float32)
    dyf = dy.astype(jnp.float32)
    g = jnp.exp((dxf * dxf + dyf * dyf) * -0.5)
    within = (jnp.abs(dx) <= TMP) & (jnp.abs(dy) <= TMP)
    tt = jnp.where(within, g, 0.0) * visf[None, None]
    diff1 = (h - tt) * m1f[None, None]
    s1 = jnp.sum(diff1 * diff1)

    @pl.when(step == 0)
    def _():
        acc_ref[0] = 0.0

    acc_ref[0] += s1

    @pl.when(step == nsteps - 1)
    def _():
        s1_ref[0] = acc_ref[0]


def _h_pass(ht, tt0, tt1, vv0):
    Rb = 128
    nsteps = B // Rb
    small = pl.BlockSpec((NJ, Rb), lambda i: (0, i))
    return pl.pallas_call(
        functools.partial(_h_body, nsteps=nsteps),
        grid=(nsteps,),
        in_specs=[
            pl.BlockSpec((COL, COL, NJ, Rb), lambda i: (0, 0, 0, i)),
            small, small, small,
        ],
        out_specs=[
            pl.BlockSpec((NJ, Rb), lambda i: (0, i)),
            pl.BlockSpec((1,), lambda i: (0,), memory_space=pltpu.SMEM),
        ],
        out_shape=[
            jax.ShapeDtypeStruct((NJ, B), jnp.int32),
            jax.ShapeDtypeStruct((1,), jnp.float32),
        ],
        scratch_shapes=[pltpu.SMEM((1,), jnp.float32)],
    )(ht, tt0, tt1, vv0)


def _sc_body(otab, idx_hbm, tT_hbm, vT_hbm, part_hbm,
             idx_v, gidx_v, gath_v, t_v, v_v, acc_v, rows_a, rows_b,
             sem_a, sem_b):
    wid = lax.axis_index("s") * 2 + lax.axis_index("c")
    base = wid * CHUNK

    pltpu.sync_copy(idx_hbm.at[pl.ds(base, CHUNK)], idx_v)
    pltpu.sync_copy(tT_hbm.at[:, pl.ds(base, CHUNK)], t_v)
    pltpu.sync_copy(vT_hbm.at[:, pl.ds(base, CHUNK)], v_v)

    iota = lax.iota(jnp.int32, 16)
    widf = wid.astype(jnp.float32)
    for m in range(NV):
        idx16f = idx_v[pl.ds(m * 16, 16)].astype(jnp.float32)
        rowbase = idx16f * 288.0 + widf * 3.0
        for c in range(3):
            row = rowbase + float(c * 96 + (m * 16) // 128)
            pos = c * CHUNK + m * 16
            gidx_v[pos // 128, pl.ds(pos % 128, 16)] = row.astype(jnp.int32)

    ngath = (3 * CHUNK) // 128
    bufs = [rows_a, rows_b]
    sems = [sem_a, sem_b]
    cps = []
    cp0 = pltpu.make_async_copy(otab.at[gidx_v.at[0]], rows_a, sems[0])
    cp0.start()
    cps.append(cp0)
    for k in range(ngath):
        if k + 1 < ngath:
            nb = bufs[(k + 1) % 2]
            cpn = pltpu.make_async_copy(otab.at[gidx_v.at[k + 1]], nb,
                                        sems[(k + 1) % 2])
            cpn.start()
            cps.append(cpn)
        cps[k].wait()
        buf = bufs[k % 2]
        for mm in range(8):
            ii = iota + mm * 16
            val = plsc.load_gather(buf, [ii, ii])
            gath_v[k, pl.ds(mm * 16, 16)] = val

    d2a = jnp.zeros((16,), jnp.float32)
    cta = jnp.zeros((16,), jnp.float32)
    n2a = jnp.zeros((16,), jnp.float32)
    for m in range(NV):
        idx16f = idx_v[pl.ds(m * 16, 16)].astype(jnp.float32)
        yCf = (idx16f * (1.0 / COL)).astype(jnp.int32).astype(jnp.float32)
        xCf = idx16f - yCf * float(COL)
        t0 = t_v[0, pl.ds(m * 16, 16)]
        t1 = t_v[1, pl.ds(m * 16, 16)]
        t2 = t_v[2, pl.ds(m * 16, 16)]
        v0 = v_v[0, pl.ds(m * 16, 16)]
        v1 = v_v[1, pl.ds(m * 16, 16)]
        v2 = v_v[2, pl.ds(m * 16, 16)]
        mu_xf = (t0 * COL + 0.5).astype(jnp.int32).astype(jnp.float32)
        mu_yf = (t1 * COL + 0.5).astype(jnp.int32).astype(jnp.float32)
        v0t = v0.astype(jnp.int32).astype(jnp.float32)
        oob = ((mu_xf - TMP >= COL) | (mu_yf - TMP >= COL)
               | (mu_xf + TMP + 1 < 0) | (mu_yf + TMP + 1 < 0))
        kill = (v0t == 1.0) & oob
        w0 = jnp.where(kill, 0.0, v0)
        w1 = jnp.where(kill, 0.0, v1)
        w2 = jnp.where(kill, 0.0, v2)
        w0t = w0.astype(jnp.int32).astype(jnp.float32)
        mask1 = jnp.where(w0t != 0.0, 1.0, 0.0)
        cta = cta + mask1
        n2a = n2a + (w0 + w1 + w2) * (1.0 / 3.0)
        gs = []
        for c in range(3):
            pos = c * CHUNK + m * 16
            gs.append(gath_v[pos // 128, pl.ds(pos % 128, 16)])
        d0 = (gs[0] + xCf * SCALE - t0) * w0
        d1 = (gs[1] + yCf * SCALE - t1) * w1
        d2 = (gs[2] - t2) * w2
        d2a = d2a + d0 * d0 + d1 * d1 + d2 * d2

    acc_v[0, :] = d2a
    acc_v[1, :] = cta
    acc_v[2, :] = n2a
    pltpu.sync_copy(acc_v, part_hbm.at[wid])


def _sc_gather(otab, idxflat, tT, vT):
    mesh = plsc.VectorSubcoreMesh(core_axis_name="c", subcore_axis_name="s")
    return pl.kernel(
        _sc_body,
        mesh=mesh,
        out_type=jax.ShapeDtypeStruct((NTILES, 3, 16), jnp.float32),
        scratch_types=[
            pltpu.VMEM((CHUNK,), jnp.int32),
            pltpu.VMEM(((3 * CHUNK) // 128, 128), jnp.int32),
            pltpu.VMEM(((3 * CHUNK) // 128, 128), jnp.float32),
            pltpu.VMEM((3, CHUNK), jnp.float32),
            pltpu.VMEM((3, CHUNK), jnp.float32),
            pltpu.VMEM((3, 16), jnp.float32),
            pltpu.VMEM((128, 128), jnp.float32),
            pltpu.VMEM((128, 128), jnp.float32),
            pltpu.SemaphoreType.DMA,
            pltpu.SemaphoreType.DMA,
        ],
        compiler_params=pltpu.CompilerParams(needs_layout_passes=False),
    )(otab, idxflat, tT, vT)


def _combine_body(part_ref, s1_ref, out_ref):
    pr = part_ref[...]
    d2 = jnp.sum(pr[:, 0, :])
    cnt = jnp.sum(pr[:, 1, :])
    n2 = jnp.sum(pr[:, 2, :])
    out_ref[0] = s1_ref[0] / cnt + d2 / n2


def _combine(partials, s1):
    return pl.pallas_call(
        _combine_body,
        in_specs=[
            pl.BlockSpec(memory_space=pltpu.VMEM),
            pl.BlockSpec(memory_space=pltpu.SMEM),
        ],
        out_specs=pl.BlockSpec(memory_space=pltpu.SMEM),
        out_shape=jax.ShapeDtypeStruct((1,), jnp.float32),
    )(partials, s1)


@jax.jit
def kernel(o, h, t, v):
    ht = h.transpose(2, 3, 1, 0)
    otab = o.transpose(2, 3, 1, 0).reshape(3 * NJ * P * B // 128, 128)
    tj = t.transpose(2, 1, 0)
    vj = v.transpose(2, 1, 0)

    idx, s1 = _h_pass(ht, tj[0], tj[1], vj[0])

    idxflat = idx.reshape(BJ)
    tT = tj.reshape(3, BJ)
    vT = vj.reshape(3, BJ)
    partials = _sc_gather(otab, idxflat, tT, vT)
    out = _combine(partials, s1)
    return out[0]


# speedup vs baseline: 1.9126x; 1.9126x over previous
import functools

import jax
import jax.numpy as jnp
from jax import lax
from jax.experimental import pallas as pl
from jax.experimental.pallas import tpu as pltpu
from jax.experimental.pallas import tpu_sc as plsc

NJ = 24
COL = 14
TMP = 3
P = COL * COL
B = 512
BJ = B * NJ
NTILES = 32
CHUNK = BJ // NTILES
NV = CHUNK // 16
SCALE = 1.0 / float(COL)


def _h_body(h_ref, t0_ref, t1_ref, v0_ref, idx_ref, s1_ref, acc_ref, *,
            nsteps):
    step = pl.program_id(0)

    h = h_ref[...]
    py = lax.broadcasted_iota(jnp.int32, h.shape, 0)
    px = lax.broadcasted_iota(jnp.int32, h.shape, 1)
    p = py * COL + px
    m = jnp.max(jnp.max(h, axis=0), axis=0)
    hit = jnp.where(h == m[None, None], p, P)
    idx = jnp.min(jnp.min(hit, axis=0), axis=0)
    idx_ref[...] = idx

    t0 = t0_ref[...]
    t1 = t1_ref[...]
    v0 = v0_ref[...]

    mu_x = (t0 * COL + 0.5).astype(jnp.int32)
    mu_y = (t1 * COL + 0.5).astype(jnp.int32)
    v0i = v0.astype(jnp.int32)
    oob = ((mu_x - TMP >= COL) | (mu_y - TMP >= COL)
           | (mu_x + TMP + 1 < 0) | (mu_y + TMP + 1 < 0))
    kill = (v0i == 1) & oob
    vis = (v0i == 1) & (~oob)
    vn0 = jnp.where(kill, 0.0, v0)
    mask1 = (vn0.astype(jnp.int32) != 0)
    m1f = mask1.astype(jnp.float32)
    visf = vis.astype(jnp.float32)

    dx = px - mu_x[None, None]
    dy = py - mu_y[None, None]
    dxf = dx.astype(jnp.float32)
    dyf = dy.astype(jnp.float32)
    g = jnp.exp((dxf * dxf + dyf * dyf) * -0.5)
    within = (jnp.abs(dx) <= TMP) & (jnp.abs(dy) <= TMP)
    tt = jnp.where(within, g, 0.0) * visf[None, None]
    diff1 = (h - tt) * m1f[None, None]
    s1 = jnp.sum(diff1 * diff1)

    @pl.when(step == 0)
    def _():
        acc_ref[0] = 0.0

    acc_ref[0] += s1

    @pl.when(step == nsteps - 1)
    def _():
        s1_ref[0] = acc_ref[0]


def _h_pass(ht, tt0, tt1, vv0):
    Rb = 128
    nsteps = B // Rb
    small = pl.BlockSpec((NJ, Rb), lambda i: (0, i))
    return pl.pallas_call(
        functools.partial(_h_body, nsteps=nsteps),
        grid=(nsteps,),
        in_specs=[
            pl.BlockSpec((COL, COL, NJ, Rb), lambda i: (0, 0, 0, i)),
            small, small, small,
        ],
        out_specs=[
            pl.BlockSpec((NJ, Rb), lambda i: (0, i)),
            pl.BlockSpec((1,), lambda i: (0,), memory_space=pltpu.SMEM),
        ],
        out_shape=[
            jax.ShapeDtypeStruct((NJ, B), jnp.int32),
            jax.ShapeDtypeStruct((1,), jnp.float32),
        ],
        scratch_shapes=[pltpu.SMEM((1,), jnp.float32)],
    )(ht, tt0, tt1, vv0)


def _sc_body(otab, idx_hbm, tT_hbm, vT_hbm, part_hbm,
             idx_v, gidx_v, gath_v, t_v, v_v, acc_v, rows_a, rows_b,
             sem_a, sem_b):
    wid = lax.axis_index("s") * 2 + lax.axis_index("c")
    base = wid * CHUNK

    pltpu.sync_copy(idx_hbm.at[pl.ds(base, CHUNK)], idx_v)
    pltpu.sync_copy(tT_hbm.at[:, pl.ds(base, CHUNK)], t_v)
    pltpu.sync_copy(vT_hbm.at[:, pl.ds(base, CHUNK)], v_v)

    iotaf = lax.iota(jnp.int32, 16).astype(jnp.float32)
    basef = base.astype(jnp.float32)
    for m in range(NV):
        n16f = basef + (m * 16.0 + iotaf)
        jf = (n16f * (1.0 / 512.0)).astype(jnp.int32).astype(jnp.float32)
        bf = n16f - jf * 512.0
        j8 = (jf * (1.0 / 8.0)).astype(jnp.int32).astype(jnp.float32)
        jm = jf - j8 * 8.0
        bq = (bf * (1.0 / 128.0)).astype(jnp.int32).astype(jnp.float32)
        bl = bf - bq * 128.0
        idx16f = idx_v[pl.ds(m * 16, 16)].astype(jnp.float32)
        rowbase = (idx16f * 36864.0 + j8 * 4096.0 + bq * 1024.0
                   + jm * 128.0 + bl)
        for c in range(3):
            fl = rowbase + float(c * BJ)
            pos = c * CHUNK + m * 16
            gidx_v[pos // 128, pl.ds(pos % 128, 16)] = fl.astype(jnp.int32)

    ngath = (3 * CHUNK) // 128
    copies = []
    for k in range(ngath):
        cp = pltpu.make_async_copy(otab.at[gidx_v.at[k]], gath_v.at[k],
                                   sem_a if k % 2 == 0 else sem_b)
        cp.start()
        copies.append(cp)
    for cp in copies:
        cp.wait()
    del rows_a, rows_b

    d2a = jnp.zeros((16,), jnp.float32)
    cta = jnp.zeros((16,), jnp.float32)
    n2a = jnp.zeros((16,), jnp.float32)
    for m in range(NV):
        idx16f = idx_v[pl.ds(m * 16, 16)].astype(jnp.float32)
        yCf = (idx16f * (1.0 / COL)).astype(jnp.int32).astype(jnp.float32)
        xCf = idx16f - yCf * float(COL)
        t0 = t_v[0, pl.ds(m * 16, 16)]
        t1 = t_v[1, pl.ds(m * 16, 16)]
        t2 = t_v[2, pl.ds(m * 16, 16)]
        v0 = v_v[0, pl.ds(m * 16, 16)]
        v1 = v_v[1, pl.ds(m * 16, 16)]
        v2 = v_v[2, pl.ds(m * 16, 16)]
        mu_xf = (t0 * COL + 0.5).astype(jnp.int32).astype(jnp.float32)
        mu_yf = (t1 * COL + 0.5).astype(jnp.int32).astype(jnp.float32)
        v0t = v0.astype(jnp.int32).astype(jnp.float32)
        oob = ((mu_xf - TMP >= COL) | (mu_yf - TMP >= COL)
               | (mu_xf + TMP + 1 < 0) | (mu_yf + TMP + 1 < 0))
        kill = (v0t == 1.0) & oob
        w0 = jnp.where(kill, 0.0, v0)
        w1 = jnp.where(kill, 0.0, v1)
        w2 = jnp.where(kill, 0.0, v2)
        w0t = w0.astype(jnp.int32).astype(jnp.float32)
        mask1 = jnp.where(w0t != 0.0, 1.0, 0.0)
        cta = cta + mask1
        n2a = n2a + (w0 + w1 + w2) * (1.0 / 3.0)
        gs = []
        for c in range(3):
            pos = c * CHUNK + m * 16
            gs.append(gath_v[pos // 128, pl.ds(pos % 128, 16)])
        d0 = (gs[0] + xCf * SCALE - t0) * w0
        d1 = (gs[1] + yCf * SCALE - t1) * w1
        d2 = (gs[2] - t2) * w2
        d2a = d2a + d0 * d0 + d1 * d1 + d2 * d2

    acc_v[0, :] = d2a
    acc_v[1, :] = cta
    acc_v[2, :] = n2a
    pltpu.sync_copy(acc_v, part_hbm.at[wid])


def _sc_gather(otab, idxflat, tT, vT):
    mesh = plsc.VectorSubcoreMesh(core_axis_name="c", subcore_axis_name="s")
    return pl.kernel(
        _sc_body,
        mesh=mesh,
        out_type=jax.ShapeDtypeStruct((NTILES, 3, 16), jnp.float32),
        scratch_types=[
            pltpu.VMEM((CHUNK,), jnp.int32),
            pltpu.VMEM(((3 * CHUNK) // 128, 128), jnp.int32),
            pltpu.VMEM(((3 * CHUNK) // 128, 128), jnp.float32),
            pltpu.VMEM((3, CHUNK), jnp.float32),
            pltpu.VMEM((3, CHUNK), jnp.float32),
            pltpu.VMEM((3, 16), jnp.float32),
            pltpu.VMEM((16,), jnp.float32),
            pltpu.VMEM((16,), jnp.float32),
            pltpu.SemaphoreType.DMA,
            pltpu.SemaphoreType.DMA,
        ],
        compiler_params=pltpu.CompilerParams(needs_layout_passes=False),
    )(otab, idxflat, tT, vT)


def _combine_body(part_ref, s1_ref, out_ref):
    pr = part_ref[...]
    d2 = jnp.sum(pr[:, 0, :])
    cnt = jnp.sum(pr[:, 1, :])
    n2 = jnp.sum(pr[:, 2, :])
    out_ref[0] = s1_ref[0] / cnt + d2 / n2


def _combine(partials, s1):
    return pl.pallas_call(
        _combine_body,
        in_specs=[
            pl.BlockSpec(memory_space=pltpu.VMEM),
            pl.BlockSpec(memory_space=pltpu.SMEM),
        ],
        out_specs=pl.BlockSpec(memory_space=pltpu.SMEM),
        out_shape=jax.ShapeDtypeStruct((1,), jnp.float32),
    )(partials, s1)


@jax.jit
def kernel(o, h, t, v):
    ht = h.transpose(2, 3, 1, 0)
    otab = (o.transpose(2, 3, 1, 0)
             .reshape(COL, COL, 3 * NJ // 8, 8, B // 128, 128)
             .transpose(0, 1, 2, 4, 3, 5)
             .reshape(-1))
    tj = t.transpose(2, 1, 0)
    vj = v.transpose(2, 1, 0)

    idx, s1 = _h_pass(ht, tj[0], tj[1], vj[0])

    idxflat = idx.reshape(BJ)
    tT = tj.reshape(3, BJ)
    vT = vj.reshape(3, BJ)
    partials = _sc_gather(otab, idxflat, tT, vT)
    out = _combine(partials, s1)
    return out[0]
